# baseline (speedup 1.0000x reference)
import functools

import jax
import jax.numpy as jnp
from jax import lax
from jax.experimental import pallas as pl
from jax.experimental.pallas import tpu as pltpu
from jax.experimental.pallas import tpu_sc as plsc

_N = 10000
_E = 320000
_D = 128
_K1 = 5000
_K2 = 2500
_STD = 0.01
_SQRT_D = 0.1

_NP0 = 10112
_NP1 = 5120
_NP2 = 2560


def _prep_body(x_ref, w_ref, deg_ref, xw_ref, y_ref, dinv_ref):
    xw = jnp.dot(x_ref[...], w_ref[...], preferred_element_type=jnp.float32)
    dinv = jax.lax.rsqrt(deg_ref[...] + 2.0)
    xw_ref[...] = xw
    y_ref[...] = xw * dinv
    dinv_ref[...] = dinv


def _prep(x, w, deg):
    n = x.shape[0]
    return pl.pallas_call(
        _prep_body,
        out_shape=(
            jax.ShapeDtypeStruct((n, _D), jnp.float32),
            jax.ShapeDtypeStruct((n, _D), jnp.float32),
            jax.ShapeDtypeStruct((n, 1), jnp.float32),
        ),
    )(x, w, deg.reshape(n, 1))


def _post_body(a0_ref, a1_ref, xw_ref, dinv_ref, b_ref, o_ref, *, do_tanh):
    dinv = dinv_ref[...]
    o = ((a0_ref[...] + a1_ref[...]) * dinv
         + 2.0 * dinv * dinv * xw_ref[...] + b_ref[...])
    o_ref[...] = jnp.tanh(o) if do_tanh else o


def _post(a0, a1, xw, dinv, b, do_tanh):
    n = a0.shape[0]
    return pl.pallas_call(
        functools.partial(_post_body, do_tanh=do_tanh),
        out_shape=jax.ShapeDtypeStruct((n, _D), jnp.float32),
    )(a0, a1, xw, dinv, b.reshape(1, _D))


def _score_body(h_ref, p_ref, s_ref):
    p = p_ref[...]
    pn = p * jax.lax.rsqrt(jnp.sum(p * p))
    s_ref[...] = jnp.tanh(jnp.sum(h_ref[...] * pn, axis=1, keepdims=True))


def _score(h, p):
    n = h.shape[0]
    s = pl.pallas_call(
        _score_body,
        out_shape=jax.ShapeDtypeStruct((n, 1), jnp.float32),
    )(h, p.reshape(1, _D))
    return s[:, 0]


_EB = 80
_E_PER_TILE = _E // 32


@functools.lru_cache(maxsize=None)
def _make_agg(n_pad):
    mesh = plsc.VectorSubcoreMesh(core_axis_name="c", subcore_axis_name="s")
    nchunks = _E_PER_TILE // _EB
    rows_per_tile = n_pad // 16
    nz = rows_per_tile // 8

    @functools.partial(
        pl.kernel, mesh=mesh,
        out_type=(jax.ShapeDtypeStruct((n_pad, _D), jnp.float32),
                  jax.ShapeDtypeStruct((n_pad, _D), jnp.float32)),
        scratch_types=[
            pltpu.VMEM((_EB,), jnp.int32),
            pltpu.VMEM((_EB,), jnp.int32),
            pltpu.VMEM((_EB, _D), jnp.float32),
            pltpu.VMEM((8, _D), jnp.float32),
            pltpu.VMEM_SHARED((n_pad, _D), jnp.float32),
            pltpu.SemaphoreType.DMA,
        ],
    )
    def agg(row_hbm, col_hbm, y_hbm, out0, out1, ridx, cidx, rows, zbuf,
            acc, sem):
        c = lax.axis_index("c")
        s = lax.axis_index("s")
        for i in range(8):
            for j in range(_D // 16):
                zbuf[i, pl.ds(j * 16, 16)] = jnp.zeros((16,), jnp.float32)
        r0 = s * rows_per_tile

        def zloop(i, carry):
            pltpu.sync_copy(zbuf, acc.at[pl.ds(r0 + i * 8, 8)])
            return carry

        lax.fori_loop(0, nz, zloop, 0)
        plsc.subcore_barrier()

        base = (c * 16 + s) * _E_PER_TILE

        def body(j, carry):
            off = base + j * _EB
            pltpu.sync_copy(row_hbm.at[pl.ds(off, _EB)], ridx)
            pltpu.sync_copy(col_hbm.at[pl.ds(off, _EB)], cidx)
            pltpu.async_copy(y_hbm.at[ridx], rows, sem).wait()
            pltpu.sync_copy(rows, acc.at[cidx], add=True)
            return carry

        lax.fori_loop(0, nchunks, body, 0)
        plsc.subcore_barrier()

        sl = pl.ds(r0, rows_per_tile)

        @pl.when(c == 0)
        def _():
            pltpu.sync_copy(acc.at[sl], out0.at[sl])

        @pl.when(c == 1)
        def _():
            pltpu.sync_copy(acc.at[sl], out1.at[sl])

    return agg


def _gcn(x, row, col, W, b, n, n_pad, do_tanh):
    deg = jnp.zeros((n + 1,), jnp.float32).at[col].add(1.0)[:n]
    xw, y, dinv = _prep(x, W, deg)
    y_p = jnp.zeros((n_pad, _D), jnp.float32).at[:n].set(y)
    p0, p1 = _make_agg(n_pad)(row, col, y_p)
    return _post(p0[:n], p1[:n], xw, dinv, b, do_tanh)


def _pool(x, row, col, score, k, n):
    vals, perm = jax.lax.top_k(score, k)
    xp = x[perm] * vals[:, None]
    inv = jnp.full((n + 1,), k, jnp.int32).at[perm].set(
        jnp.arange(k, dtype=jnp.int32))
    r = inv[row]
    c = inv[col]
    invalid = (r == k) | (c == k)
    nrow = jnp.where(invalid, k, r)
    ncol = jnp.where(invalid, k, c)
    return xp, nrow, ncol, perm


def kernel(x, edge_index, W_down0, b_down0, W_down1, b_down1, W_down2, b_down2,
           p_pool1, p_pool2, W_up1, b_up1, W_up2, b_up2):
    row = edge_index[0]
    col = edge_index[1]

    h = _gcn(x, row, col, W_down0, b_down0, _N, _NP0, True)
    x0 = h
    s1 = _score(h, p_pool1)
    h, r1, c1, perm1 = _pool(h, row, col, s1, _K1, _N)
    h = _gcn(h, r1, c1, W_down1, b_down1, _K1, _NP1, True)
    x1 = h
    s2 = _score(h, p_pool2)
    h, r2, c2, perm2 = _pool(h, r1, c1, s2, _K2, _K1)
    h = _gcn(h, r2, c2, W_down2, b_down2, _K2, _NP2, True)

    up = jnp.zeros_like(x1).at[perm2].set(h)
    h = x1 + up
    h = _gcn(h, r1, c1, W_up1, b_up1, _K1, _NP1, True)
    up = jnp.zeros_like(x0).at[perm1].set(h)
    h = x0 + up
    drift = _gcn(h, row, col, W_up2, b_up2, _N, _NP0, False)

    z = jax.random.normal(jax.random.fold_in(jax.random.key(0), 777),
                          drift.shape, dtype=drift.dtype)
    return drift + _STD * z / _SQRT_D


# speedup vs baseline: 3.5827x; 3.5827x over previous
import functools

import jax
import jax.numpy as jnp
from jax import lax
from jax.experimental import pallas as pl
from jax.experimental.pallas import tpu as pltpu
from jax.experimental.pallas import tpu_sc as plsc

_N = 10000
_E = 320000
_D = 128
_K1 = 5000
_K2 = 2500
_STD = 0.01
_SQRT_D = 0.1

_NP0 = 10112
_NP1 = 5120
_NP2 = 2560


def _prep_body(x_ref, w_ref, deg_ref, xw_ref, y_ref, dinv_ref):
    xw = jnp.dot(x_ref[...], w_ref[...], preferred_element_type=jnp.float32)
    dinv = jax.lax.rsqrt(deg_ref[...] + 2.0)
    xw_ref[...] = xw
    y_ref[...] = xw * dinv
    dinv_ref[...] = dinv


def _prep(x, w, deg):
    n = x.shape[0]
    return pl.pallas_call(
        _prep_body,
        out_shape=(
            jax.ShapeDtypeStruct((n, _D), jnp.float32),
            jax.ShapeDtypeStruct((n, _D), jnp.float32),
            jax.ShapeDtypeStruct((n, 1), jnp.float32),
        ),
    )(x, w, deg.reshape(n, 1))


def _post_body(a0_ref, a1_ref, xw_ref, dinv_ref, b_ref, o_ref, *, do_tanh):
    dinv = dinv_ref[...]
    o = ((a0_ref[...] + a1_ref[...]) * dinv
         + 2.0 * dinv * dinv * xw_ref[...] + b_ref[...])
    o_ref[...] = jnp.tanh(o) if do_tanh else o


def _post(a0, a1, xw, dinv, b, do_tanh):
    n = a0.shape[0]
    return pl.pallas_call(
        functools.partial(_post_body, do_tanh=do_tanh),
        out_shape=jax.ShapeDtypeStruct((n, _D), jnp.float32),
    )(a0, a1, xw, dinv, b.reshape(1, _D))


def _score_body(h_ref, p_ref, s_ref):
    p = p_ref[...]
    pn = p * jax.lax.rsqrt(jnp.sum(p * p))
    s_ref[...] = jnp.tanh(jnp.sum(h_ref[...] * pn, axis=1, keepdims=True))


def _score(h, p):
    n = h.shape[0]
    s = pl.pallas_call(
        _score_body,
        out_shape=jax.ShapeDtypeStruct((n, 1), jnp.float32),
    )(h, p.reshape(1, _D))
    return s[:, 0]


_EB = 80
_E_PER_TILE = _E // 32


@functools.lru_cache(maxsize=None)
def _make_agg(n_pad):
    mesh = plsc.VectorSubcoreMesh(core_axis_name="c", subcore_axis_name="s")
    nchunks = _E_PER_TILE // _EB
    rows_per_tile = n_pad // 16
    nz = rows_per_tile // 8

    @functools.partial(
        pl.kernel, mesh=mesh,
        out_type=(jax.ShapeDtypeStruct((n_pad, _D), jnp.float32),
                  jax.ShapeDtypeStruct((n_pad, _D), jnp.float32)),
        scratch_types=[
            pltpu.VMEM((_EB,), jnp.int32),
            pltpu.VMEM((_EB,), jnp.int32),
            pltpu.VMEM((_EB, _D), jnp.float32),
            pltpu.VMEM((8, _D), jnp.float32),
            pltpu.VMEM_SHARED((n_pad, _D), jnp.float32),
            pltpu.SemaphoreType.DMA,
        ],
    )
    def agg(row_hbm, col_hbm, y_hbm, out0, out1, ridx, cidx, rows, zbuf,
            acc, sem):
        c = lax.axis_index("c")
        s = lax.axis_index("s")
        for i in range(8):
            for j in range(_D // 16):
                zbuf[i, pl.ds(j * 16, 16)] = jnp.zeros((16,), jnp.float32)
        r0 = s * rows_per_tile

        def zloop(i, carry):
            pltpu.sync_copy(zbuf, acc.at[pl.ds(r0 + i * 8, 8)])
            return carry

        lax.fori_loop(0, nz, zloop, 0)
        plsc.subcore_barrier()

        base = (c * 16 + s) * _E_PER_TILE

        def body(j, carry):
            off = base + j * _EB
            pltpu.sync_copy(row_hbm.at[pl.ds(off, _EB)], ridx)
            pltpu.sync_copy(col_hbm.at[pl.ds(off, _EB)], cidx)
            pltpu.async_copy(y_hbm.at[ridx], rows, sem).wait()
            pltpu.sync_copy(rows, acc.at[cidx], add=True)
            return carry

        lax.fori_loop(0, nchunks, body, 0)
        plsc.subcore_barrier()

        sl = pl.ds(r0, rows_per_tile)

        @pl.when(c == 0)
        def _():
            pltpu.sync_copy(acc.at[sl], out0.at[sl])

        @pl.when(c == 1)
        def _():
            pltpu.sync_copy(acc.at[sl], out1.at[sl])

    return agg


def _gcn(x, row, col, W, b, n, n_pad, do_tanh):
    deg = jnp.zeros((n_pad,), jnp.float32).at[col].add(1.0)[:n]
    xw, y, dinv = _prep(x, W, deg)
    y_p = jnp.zeros((n_pad, _D), jnp.float32).at[:n].set(y)
    p0, p1 = _make_agg(n_pad)(row, col, y_p)
    return _post(p0[:n], p1[:n], xw, dinv, b, do_tanh)


def _pool(x, row, col, score, k, n, n_pad):
    vals, perm = jax.lax.top_k(score, k)
    xp = x[perm] * vals[:, None]
    inv = jnp.full((n + 1,), k, jnp.int32).at[perm].set(
        jnp.arange(k, dtype=jnp.int32))
    r = inv[row]
    c = inv[col]
    invalid = (r == k) | (c == k)
    spread = k + jnp.arange(_E, dtype=jnp.int32) % (n_pad - k)
    nrow = jnp.where(invalid, spread, r)
    ncol = jnp.where(invalid, spread, c)
    return xp, nrow, ncol, perm


def kernel(x, edge_index, W_down0, b_down0, W_down1, b_down1, W_down2, b_down2,
           p_pool1, p_pool2, W_up1, b_up1, W_up2, b_up2):
    row = edge_index[0]
    col = edge_index[1]

    h = _gcn(x, row, col, W_down0, b_down0, _N, _NP0, True)
    x0 = h
    s1 = _score(h, p_pool1)
    h, r1, c1, perm1 = _pool(h, row, col, s1, _K1, _N, _NP1)
    h = _gcn(h, r1, c1, W_down1, b_down1, _K1, _NP1, True)
    x1 = h
    s2 = _score(h, p_pool2)
    h, r2, c2, perm2 = _pool(h, r1, c1, s2, _K2, _K1, _NP2)
    h = _gcn(h, r2, c2, W_down2, b_down2, _K2, _NP2, True)

    up = jnp.zeros_like(x1).at[perm2].set(h)
    h = x1 + up
    h = _gcn(h, r1, c1, W_up1, b_up1, _K1, _NP1, True)
    up = jnp.zeros_like(x0).at[perm1].set(h)
    h = x0 + up
    drift = _gcn(h, row, col, W_up2, b_up2, _N, _NP0, False)

    z = jax.random.normal(jax.random.fold_in(jax.random.key(0), 777),
                          drift.shape, dtype=drift.dtype)
    return drift + _STD * z / _SQRT_D


# speedup vs baseline: 12.5313x; 3.4977x over previous
import functools

import jax
import jax.numpy as jnp
from jax import lax
from jax.experimental import pallas as pl
from jax.experimental.pallas import tpu as pltpu
from jax.experimental.pallas import tpu_sc as plsc

_N = 10000
_E = 320000
_D = 128
_K1 = 5000
_K2 = 2500
_STD = 0.01
_SQRT_D = 0.1

_NP0 = 10240
_NP1 = 5120
_NP2 = 2560

_EB = 80
_E_PER_TILE = _E // 32


def _prep_body(a_ref, b_ref, w_ref, d0_ref, d1_ref, xw_ref, y_ref,
               dinv_ref, *, n, mode):
    if mode == "x0":
        xin = a_ref[...]
    elif mode == "up":
        xin = a_ref[...] + b_ref[...]
    else:
        xin = a_ref[...]
    xw = jnp.dot(xin, w_ref[...], preferred_element_type=jnp.float32)
    if mode == "pool":
        xw = xw * b_ref[...]
    dinv = jax.lax.rsqrt(d0_ref[...] + d1_ref[...] + 2.0)
    n_pad = dinv_ref.shape[0]
    if mode == "x0":
        xw_ref[:n, :] = xw
        xw_ref[n:, :] = jnp.zeros((n_pad - n, _D), jnp.float32)
        y_ref[:n, :] = xw * dinv[:n]
        y_ref[n:, :] = jnp.zeros((n_pad - n, _D), jnp.float32)
    else:
        ri = lax.broadcasted_iota(jnp.int32, (n_pad, 1), 0)
        xw = jnp.where(ri < n, xw, 0.0)
        xw_ref[...] = xw
        y_ref[...] = xw * dinv
    dinv_ref[...] = dinv


def _prep(a, b, w, d0, d1, n, n_pad, mode):
    body = functools.partial(_prep_body, n=n, mode=mode)
    args = [a]
    if mode == "up":
        args.append(b)
    elif mode == "pool":
        args.append(b.reshape(n_pad, 1))
    else:
        args.append(jnp.zeros((1, 1), jnp.float32))
    args += [w, d0.reshape(n_pad, 1), d1.reshape(n_pad, 1)]
    return pl.pallas_call(
        body,
        out_shape=(
            jax.ShapeDtypeStruct((n_pad, _D), jnp.float32),
            jax.ShapeDtypeStruct((n_pad, _D), jnp.float32),
            jax.ShapeDtypeStruct((n_pad, 1), jnp.float32),
        ),
    )(*args)


def _post_body(p0_ref, p1_ref, xw_ref, dinv_ref, b_ref, o_ref, *, n,
               do_tanh):
    dinv = dinv_ref[...]
    o = ((p0_ref[...] + p1_ref[...]) * dinv
         + 2.0 * dinv * dinv * xw_ref[...] + b_ref[...])
    if do_tanh:
        o = jnp.tanh(o)
    ri = lax.broadcasted_iota(jnp.int32, o.shape, 0)
    o_ref[...] = jnp.where(ri < n, o, 0.0)


def _post(p0, p1, xw, dinv, b, n, do_tanh):
    n_pad = p0.shape[0]
    return pl.pallas_call(
        functools.partial(_post_body, n=n, do_tanh=do_tanh),
        out_shape=jax.ShapeDtypeStruct((n_pad, _D), jnp.float32),
    )(p0, p1, xw, dinv, b.reshape(1, _D))


def _post_final_body(p0_ref, p1_ref, xw_ref, dinv_ref, b_ref, z_ref, o_ref):
    dinv = dinv_ref[...]
    o = ((p0_ref[...] + p1_ref[...]) * dinv
         + 2.0 * dinv * dinv * xw_ref[...] + b_ref[...])
    o_ref[...] = o[:_N, :] + z_ref[...]


def _post_final(p0, p1, xw, dinv, b, z):
    return pl.pallas_call(
        _post_final_body,
        out_shape=jax.ShapeDtypeStruct((_N, _D), jnp.float32),
    )(p0, p1, xw, dinv, b.reshape(1, _D), z)


def _post_pool_body(p0_ref, p1_ref, xw_ref, dinv_ref, b_ref, pv_ref,
                    h_ref, s_ref, sel_ref, *, n, k):
    dinv = dinv_ref[...]
    h = jnp.tanh((p0_ref[...] + p1_ref[...]) * dinv
                 + 2.0 * dinv * dinv * xw_ref[...] + b_ref[...])
    n_pad = h.shape[0]
    ri = lax.broadcasted_iota(jnp.int32, (n_pad, 1), 0)
    h = jnp.where(ri < n, h, 0.0)
    h_ref[...] = h
    pv = pv_ref[...]
    attn = jnp.sum(h * pv, axis=1, keepdims=True) / jnp.sqrt(
        jnp.sum(pv * pv))
    score = jnp.tanh(attn)
    score = jnp.where(ri < n, score, -2.0)
    s_ref[...] = score

    key = jax.lax.bitcast_convert_type(score, jnp.int32)
    key = jnp.where(key >= 0, key, key ^ jnp.int32(0x7FFFFFFF))
    cnt_nn = jnp.sum((key >= 0).astype(jnp.int32))
    lo = jnp.where(cnt_nn >= k, jnp.int32(0), jnp.int32(-2**31))
    hi = jnp.where(cnt_nn >= k, jnp.int32(2**31 - 1), jnp.int32(-1))

    def bs1(_, c):
        lo, hi = c
        mid = lo + (hi - lo) // 2
        pred = jnp.sum((key >= mid + 1).astype(jnp.int32)) >= k
        return (jnp.where(pred, mid + 1, lo), jnp.where(pred, hi, mid))

    lo, hi = lax.fori_loop(0, 31, bs1, (lo, hi))
    t = lo
    tie = key == t
    r = k - jnp.sum((key > t).astype(jnp.int32))

    def bs2(_, c):
        lo, hi = c
        mid = lo + (hi - lo) // 2
        pred = jnp.sum((tie & (ri <= mid)).astype(jnp.int32)) >= r
        return (jnp.where(pred, lo, mid + 1), jnp.where(pred, mid, hi))

    lo2, hi2 = lax.fori_loop(0, 14, bs2,
                             (jnp.int32(0), jnp.int32(n_pad - 1)))
    sel = (key > t) | (tie & (ri <= lo2))
    sel_ref[...] = sel.astype(jnp.int32)


def _post_pool(p0, p1, xw, dinv, b, pv, n, k):
    n_pad = p0.shape[0]
    return pl.pallas_call(
        functools.partial(_post_pool_body, n=n, k=k),
        out_shape=(
            jax.ShapeDtypeStruct((n_pad, _D), jnp.float32),
            jax.ShapeDtypeStruct((n_pad, 1), jnp.float32),
            jax.ShapeDtypeStruct((n_pad, 1), jnp.int32),
        ),
    )(p0, p1, xw, dinv, b.reshape(1, _D), pv.reshape(1, _D))


@functools.lru_cache(maxsize=None)
def _mesh():
    return plsc.VectorSubcoreMesh(core_axis_name="c", subcore_axis_name="s")


_GDN = lax.GatherDimensionNumbers(
    offset_dims=(), collapsed_slice_dims=(0,), start_index_map=(0,))


def _vtake(v, idx):
    return lax.gather(v, idx[:, None], _GDN, (1,),
                      mode=lax.GatherScatterMode.PROMISE_IN_BOUNDS)


def _vcumsum(v):
    iota = jax.lax.iota(jnp.int32, 16)
    for sh in (1, 2, 4, 8):
        idx = jnp.maximum(iota - sh, 0)
        v = v + jnp.where(iota >= sh, _vtake(v, idx), 0)
    return v


@functools.lru_cache(maxsize=None)
def _make_agg(n_pad):
    nchunks = _E_PER_TILE // _EB
    rows_per_tile = n_pad // 16
    nz = rows_per_tile // 8

    @functools.partial(
        pl.kernel, mesh=_mesh(),
        out_type=(jax.ShapeDtypeStruct((n_pad, _D), jnp.float32),
                  jax.ShapeDtypeStruct((n_pad, _D), jnp.float32)),
        scratch_types=[
            pltpu.VMEM((_EB,), jnp.int32),
            pltpu.VMEM((_EB,), jnp.int32),
            pltpu.VMEM((_EB, _D), jnp.float32),
            pltpu.VMEM((8, _D), jnp.float32),
            pltpu.VMEM_SHARED((n_pad, _D), jnp.float32),
            pltpu.SemaphoreType.DMA,
        ],
    )
    def agg(row_hbm, col_hbm, y_hbm, out0, out1, ridx, cidx, rows, zbuf,
            acc, sem):
        c = lax.axis_index("c")
        s = lax.axis_index("s")
        for i in range(8):
            for j in range(_D // 16):
                zbuf[i, pl.ds(j * 16, 16)] = jnp.zeros((16,), jnp.float32)
        r0 = s * rows_per_tile

        def zloop(i, carry):
            pltpu.sync_copy(zbuf, acc.at[pl.ds(r0 + i * 8, 8)])
            return carry

        lax.fori_loop(0, nz, zloop, 0)
        plsc.subcore_barrier()

        base = (c * 16 + s) * _E_PER_TILE

        def body(j, carry):
            off = base + j * _EB
            pltpu.sync_copy(row_hbm.at[pl.ds(off, _EB)], ridx)
            pltpu.sync_copy(col_hbm.at[pl.ds(off, _EB)], cidx)
            pltpu.async_copy(y_hbm.at[ridx], rows, sem).wait()
            pltpu.sync_copy(rows, acc.at[cidx], add=True)
            return carry

        lax.fori_loop(0, nchunks, body, 0)
        plsc.subcore_barrier()

        sl = pl.ds(r0, rows_per_tile)

        @pl.when(c == 0)
        def _():
            pltpu.sync_copy(acc.at[sl], out0.at[sl])

        @pl.when(c == 1)
        def _():
            pltpu.sync_copy(acc.at[sl], out1.at[sl])

    return agg


@functools.lru_cache(maxsize=None)
def _make_deg0(n_pad):
    nchunks = _E_PER_TILE // _EB
    chunk_m = n_pad // 16
    nzv = chunk_m // 16

    @functools.partial(
        pl.kernel, mesh=_mesh(),
        out_type=(jax.ShapeDtypeStruct((n_pad,), jnp.float32),
                  jax.ShapeDtypeStruct((n_pad,), jnp.float32)),
        scratch_types=[
            pltpu.VMEM((_EB,), jnp.int32),
            pltpu.VMEM((_EB,), jnp.float32),
            pltpu.VMEM((chunk_m,), jnp.float32),
            pltpu.VMEM_SHARED((n_pad,), jnp.float32),
        ],
    )
    def deg0(col_hbm, out0, out1, cidx, ones, zdeg, deg_sh):
        c = lax.axis_index("c")
        s = lax.axis_index("s")
        for v in range(_EB // 16):
            ones[pl.ds(v * 16, 16)] = jnp.ones((16,), jnp.float32)
        for v in range(nzv):
            zdeg[pl.ds(v * 16, 16)] = jnp.zeros((16,), jnp.float32)
        mbase = s * chunk_m
        pltpu.sync_copy(zdeg, deg_sh.at[pl.ds(mbase, chunk_m)])
        plsc.subcore_barrier()

        base = (c * 16 + s) * _E_PER_TILE

        def body(j, carry):
            off = base + j * _EB
            pltpu.sync_copy(col_hbm.at[pl.ds(off, _EB)], cidx)
            pltpu.sync_copy(ones, deg_sh.at[cidx], add=True)
            return carry

        lax.fori_loop(0, nchunks, body, 0)
        plsc.subcore_barrier()

        sl = pl.ds(mbase, chunk_m)
        pltpu.sync_copy(deg_sh.at[sl], zdeg)

        @pl.when(c == 0)
        def _():
            pltpu.sync_copy(zdeg, out0.at[sl])

        @pl.when(c == 1)
        def _():
            pltpu.sync_copy(zdeg, out1.at[sl])

    return deg0


@functools.lru_cache(maxsize=None)
def _make_pool(np_prev, n_pad, k, n_prev):
    chunk_n = np_prev // 16
    nv = chunk_n // 16
    nv8 = chunk_n // _EB
    chunk_m = n_pad // 16
    nzv = chunk_m // 16
    rows_out = n_pad // 16
    npk = n_pad - k
    nchunks = _E_PER_TILE // _EB

    @functools.partial(
        pl.kernel, mesh=_mesh(),
        compiler_params=pltpu.CompilerParams(needs_layout_passes=False),
        out_type=(
            jax.ShapeDtypeStruct((np_prev,), jnp.int32),
            jax.ShapeDtypeStruct((n_pad, _D), jnp.float32),
            jax.ShapeDtypeStruct((n_pad,), jnp.float32),
            jax.ShapeDtypeStruct((_E,), jnp.int32),
            jax.ShapeDtypeStruct((_E,), jnp.int32),
            jax.ShapeDtypeStruct((n_pad,), jnp.float32),
            jax.ShapeDtypeStruct((n_pad,), jnp.float32),
        ),
        scratch_types=[
            pltpu.VMEM((chunk_n,), jnp.int32),
            pltpu.VMEM((chunk_n,), jnp.float32),
            pltpu.VMEM((chunk_n,), jnp.int32),
            pltpu.VMEM((chunk_n,), jnp.int32),
            pltpu.VMEM((chunk_n,), jnp.float32),
            pltpu.VMEM((nv8, _EB), jnp.int32),
            pltpu.VMEM((np_prev,), jnp.int32),
            pltpu.VMEM((_EB, _D), jnp.float32),
            pltpu.VMEM((np_prev,), jnp.int32),
            pltpu.VMEM((_EB,), jnp.int32),
            pltpu.VMEM((_EB,), jnp.int32),
            pltpu.VMEM((_EB,), jnp.int32),
            pltpu.VMEM((_EB,), jnp.int32),
            pltpu.VMEM((_EB,), jnp.float32),
            pltpu.VMEM((chunk_m,), jnp.float32),
            pltpu.VMEM((rows_out,), jnp.float32),
            pltpu.VMEM_SHARED((n_pad, _D), jnp.float32),
            pltpu.VMEM_SHARED((n_pad,), jnp.float32),
            pltpu.VMEM_SHARED((np_prev,), jnp.int32),
            pltpu.VMEM_SHARED((n_pad,), jnp.float32),
            pltpu.SemaphoreType.DMA,
        ],
    )
    def pool(sel_hbm, score_hbm, h_hbm, row_hbm, col_hbm,
             inv_out, xp_out, vals_out, nrow_out, ncol_out, dg0, dg1,
             selbuf, sbuf, invbuf, pbuf, svbuf, rankbuf, selall,
             hrows, inv_local, ebuf_r, ebuf_c, obuf_r, obuf_c, ones,
             zdeg, vbounce, xp_sh, vals_sh, inv_sh, deg_sh,
             sem):
        c = lax.axis_index("c")
        s = lax.axis_index("s")
        iota = jax.lax.iota(jnp.int32, 16)

        base = s * chunk_n
        pltpu.sync_copy(sel_hbm.at[pl.ds(base, chunk_n)], selbuf)
        pltpu.sync_copy(score_hbm.at[pl.ds(base, chunk_n)], sbuf)
        pltpu.sync_copy(sel_hbm, selall)
        fifteen = jnp.full((16,), 15, jnp.int32)

        off = jnp.zeros((16,), jnp.int32)
        run = jnp.zeros((16,), jnp.int32)
        for j in range(16):
            off = jnp.where(jnp.int32(j) == s, run, off)
            acc = jnp.zeros((16,), jnp.int32)
            for v in range(nv):
                acc = acc + selall[pl.ds(j * chunk_n + v * 16, 16)]
            run = run + _vtake(_vcumsum(acc), fifteen)

        for v in range(nzv):
            zdeg[pl.ds(v * 16, 16)] = jnp.zeros((16,), jnp.float32)
        mbase = s * chunk_m
        pltpu.sync_copy(zdeg, deg_sh.at[pl.ds(mbase, chunk_m)])
        for v in range(_EB // 16):
            ones[pl.ds(v * 16, 16)] = jnp.ones((16,), jnp.float32)

        for v in range(nv):
            pbuf[pl.ds(v * 16, 16)] = jnp.full((16,), n_prev, jnp.int32)
        loc = jnp.zeros((16,), jnp.int32)
        for v in range(nv):
            selv = selbuf[pl.ds(v * 16, 16)]
            m = selv > 0
            cum = _vcumsum(selv)
            pos = loc + cum - 1
            invbuf[pl.ds(v * 16, 16)] = jnp.where(m, off + pos, k)
            gidx = base + v * 16 + iota
            plsc.store_scatter(pbuf, [pos], gidx, mask=m)
            plsc.store_scatter(svbuf, [pos], sbuf[pl.ds(v * 16, 16)],
                               mask=m)
            loc = loc + _vtake(cum, fifteen)
        for ci in range(nv8):
            for v in range(_EB // 16):
                jj = ci * _EB + v * 16 + iota
                rankbuf[ci, pl.ds(v * 16, 16)] = jnp.where(
                    jj < loc, off + jj, k)

        pltpu.sync_copy(invbuf, inv_sh.at[pl.ds(base, chunk_n)])

        @pl.when(c == 0)
        def _():
            pltpu.sync_copy(invbuf, inv_out.at[pl.ds(base, chunk_n)])

        for ci in range(nv8):
            pltpu.async_copy(h_hbm.at[pbuf.at[pl.ds(ci * _EB, _EB)]],
                             hrows, sem).wait()
            pltpu.sync_copy(hrows, xp_sh.at[rankbuf.at[ci]])
            pltpu.sync_copy(svbuf.at[pl.ds(ci * _EB, _EB)],
                            vals_sh.at[rankbuf.at[ci]])
        plsc.subcore_barrier()

        pltpu.sync_copy(inv_sh, inv_local)
        ebase = (c * 16 + s) * _E_PER_TILE

        def body(j, carry):
            off_e = ebase + j * _EB
            pltpu.sync_copy(row_hbm.at[pl.ds(off_e, _EB)], ebuf_r)
            pltpu.sync_copy(col_hbm.at[pl.ds(off_e, _EB)], ebuf_c)
            for v in range(_EB // 16):
                rv = ebuf_r[pl.ds(v * 16, 16)]
                cv = ebuf_c[pl.ds(v * 16, 16)]
                rr = plsc.load_gather(inv_local, [rv])
                cc = plsc.load_gather(inv_local, [cv])
                bad = (rr >= k) | (cc >= k)
                obuf_r[pl.ds(v * 16, 16)] = jnp.where(
                    bad, k + lax.rem(rv, jnp.int32(npk)), rr)
                obuf_c[pl.ds(v * 16, 16)] = jnp.where(
                    bad, k + lax.rem(cv, jnp.int32(npk)), cc)
            pltpu.sync_copy(obuf_r, nrow_out.at[pl.ds(off_e, _EB)])
            pltpu.sync_copy(obuf_c, ncol_out.at[pl.ds(off_e, _EB)])
            pltpu.sync_copy(ones, deg_sh.at[obuf_c], add=True)
            return carry

        lax.fori_loop(0, nchunks, body, 0)
        plsc.subcore_barrier()

        msl = pl.ds(mbase, chunk_m)
        pltpu.sync_copy(deg_sh.at[msl], zdeg)

        @pl.when(c == 0)
        def _():
            pltpu.sync_copy(zdeg, dg0.at[msl])
            rsl = pl.ds(s * rows_out, rows_out)
            pltpu.sync_copy(xp_sh.at[rsl], xp_out.at[rsl])
            pltpu.sync_copy(vals_sh.at[rsl], vbounce)
            pltpu.sync_copy(vbounce, vals_out.at[rsl])

        @pl.when(c == 1)
        def _():
            pltpu.sync_copy(zdeg, dg1.at[msl])

    return pool


@functools.lru_cache(maxsize=None)
def _make_unpool(out_pad, src_pad):
    rpt = out_pad // 32
    nchunks = rpt // _EB

    @functools.partial(
        pl.kernel, mesh=_mesh(),
        out_type=jax.ShapeDtypeStruct((out_pad, _D), jnp.float32),
        scratch_types=[
            pltpu.VMEM((_EB,), jnp.int32),
            pltpu.VMEM((_EB, _D), jnp.float32),
            pltpu.SemaphoreType.DMA,
        ],
    )
    def unpool(inv_hbm, src_hbm, out, ibuf, rows, sem):
        c = lax.axis_index("c")
        s = lax.axis_index("s")
        wbase = (c * 16 + s) * rpt

        def body(j, carry):
            off = wbase + j * _EB
            pltpu.sync_copy(inv_hbm.at[pl.ds(off, _EB)], ibuf)
            pltpu.async_copy(src_hbm.at[ibuf], rows, sem).wait()
            pltpu.sync_copy(rows, out.at[pl.ds(off, _EB)])
            return carry

        lax.fori_loop(0, nchunks, body, 0)

    return unpool


def kernel(x, edge_index, W_down0, b_down0, W_down1, b_down1, W_down2,
           b_down2, p_pool1, p_pool2, W_up1, b_up1, W_up2, b_up2):
    row = edge_index[0]
    col = edge_index[1]
    zz = (_STD / _SQRT_D) * jax.random.normal(
        jax.random.fold_in(jax.random.key(0), 777), (_N, _D), jnp.float32)

    d0a, d0b = _make_deg0(_NP0)(col)
    xw, y, dinv = _prep(x, None, W_down0, d0a, d0b, _N, _NP0, "x0")
    p0, p1 = _make_agg(_NP0)(row, col, y)
    h0, s1, sel1 = _post_pool(p0, p1, xw, dinv, b_down0, p_pool1, _N, _K1)

    inv1, xp1, vals1, r1, c1, d1a, d1b = _make_pool(_NP0, _NP1, _K1, _N)(
        sel1.reshape(_NP0), s1.reshape(_NP0), h0, row, col)
    xw, y, dinv1 = _prep(xp1, vals1, W_down1, d1a, d1b, _K1, _NP1, "pool")
    p0, p1 = _make_agg(_NP1)(r1, c1, y)
    h1, s2, sel2 = _post_pool(p0, p1, xw, dinv1, b_down1, p_pool2, _K1, _K2)

    inv2, xp2, vals2, r2, c2, d2a, d2b = _make_pool(_NP1, _NP2, _K2, _K1)(
        sel2.reshape(_NP1), s2.reshape(_NP1), h1, r1, c1)
    xw, y, dinv2 = _prep(xp2, vals2, W_down2, d2a, d2b, _K2, _NP2, "pool")
    p0, p1 = _make_agg(_NP2)(r2, c2, y)
    h2 = _post(p0, p1, xw, dinv2, b_down2, _K2, True)

    up1 = _make_unpool(_NP1, _NP2)(inv2, h2)
    xw, y, dinvu = _prep(h1, up1, W_up1, d1a, d1b, _K1, _NP1, "up")
    p0, p1 = _make_agg(_NP1)(r1, c1, y)
    hu1 = _post(p0, p1, xw, dinvu, b_up1, _K1, True)

    up0 = _make_unpool(_NP0, _NP1)(inv1, hu1)
    xw, y, dinvu = _prep(h0, up0, W_up2, d0a, d0b, _N, _NP0, "up")
    p0, p1 = _make_agg(_NP0)(row, col, y)
    return _post_final(p0, p1, xw, dinvu, b_up2, zz)


# speedup vs baseline: 14.2952x; 1.1408x over previous
import functools

import jax
import jax.numpy as jnp
from jax import lax
from jax.experimental import pallas as pl
from jax.experimental.pallas import tpu as pltpu
from jax.experimental.pallas import tpu_sc as plsc

_N = 10000
_E = 320000
_D = 128
_K1 = 5000
_K2 = 2500
_STD = 0.01
_SQRT_D = 0.1

_NP0 = 10240
_NP1 = 5120
_NP2 = 2560

_EB = 80
_E_PAD = 327680
_EBE = 128
_E_PER_TILE = _E_PAD // 32


def _prep_body(a_ref, b_ref, w_ref, d0_ref, d1_ref, xw_ref, y_ref,
               dinv_ref, *, n, mode):
    if mode == "x0":
        xin = a_ref[...]
    elif mode == "up":
        xin = a_ref[...] + b_ref[...]
    else:
        xin = a_ref[...]
    xw = jnp.dot(xin, w_ref[...], preferred_element_type=jnp.float32)
    if mode == "pool":
        xw = xw * b_ref[...]
    dinv = jax.lax.rsqrt(d0_ref[...] + d1_ref[...] + 2.0)
    n_pad = dinv_ref.shape[0]
    if mode == "x0":
        xw_ref[:n, :] = xw
        xw_ref[n:, :] = jnp.zeros((n_pad - n, _D), jnp.float32)
        y_ref[:n, :] = xw * dinv[:n]
        y_ref[n:, :] = jnp.zeros((n_pad - n, _D), jnp.float32)
    else:
        ri = lax.broadcasted_iota(jnp.int32, (n_pad, 1), 0)
        xw = jnp.where(ri < n, xw, 0.0)
        xw_ref[...] = xw
        y_ref[...] = xw * dinv
    dinv_ref[...] = dinv


def _prep(a, b, w, d0, d1, n, n_pad, mode):
    body = functools.partial(_prep_body, n=n, mode=mode)
    args = [a]
    if mode == "up":
        args.append(b)
    elif mode == "pool":
        args.append(b.reshape(n_pad, 1))
    else:
        args.append(jnp.zeros((1, 1), jnp.float32))
    args += [w, d0.reshape(n_pad, 1), d1.reshape(n_pad, 1)]
    return pl.pallas_call(
        body,
        out_shape=(
            jax.ShapeDtypeStruct((n_pad, _D), jnp.float32),
            jax.ShapeDtypeStruct((n_pad, _D), jnp.float32),
            jax.ShapeDtypeStruct((n_pad, 1), jnp.float32),
        ),
    )(*args)


def _post_body(p0_ref, p1_ref, xw_ref, dinv_ref, b_ref, o_ref, *, n,
               do_tanh):
    dinv = dinv_ref[...]
    o = ((p0_ref[...] + p1_ref[...]) * dinv
         + 2.0 * dinv * dinv * xw_ref[...] + b_ref[...])
    if do_tanh:
        o = jnp.tanh(o)
    ri = lax.broadcasted_iota(jnp.int32, o.shape, 0)
    o_ref[...] = jnp.where(ri < n, o, 0.0)


def _post(p0, p1, xw, dinv, b, n, do_tanh):
    n_pad = p0.shape[0]
    return pl.pallas_call(
        functools.partial(_post_body, n=n, do_tanh=do_tanh),
        out_shape=jax.ShapeDtypeStruct((n_pad, _D), jnp.float32),
    )(p0, p1, xw, dinv, b.reshape(1, _D))


def _post_final_body(p0_ref, p1_ref, xw_ref, dinv_ref, b_ref, z_ref, o_ref):
    dinv = dinv_ref[...]
    o = ((p0_ref[...] + p1_ref[...]) * dinv
         + 2.0 * dinv * dinv * xw_ref[...] + b_ref[...])
    o_ref[...] = o[:_N, :] + z_ref[...]


def _post_final(p0, p1, xw, dinv, b, z):
    return pl.pallas_call(
        _post_final_body,
        out_shape=jax.ShapeDtypeStruct((_N, _D), jnp.float32),
    )(p0, p1, xw, dinv, b.reshape(1, _D), z)


def _post_pool_body(p0_ref, p1_ref, xw_ref, dinv_ref, b_ref, pv_ref,
                    h_ref, s_ref, sel_ref, *, n, k):
    dinv = dinv_ref[...]
    h = jnp.tanh((p0_ref[...] + p1_ref[...]) * dinv
                 + 2.0 * dinv * dinv * xw_ref[...] + b_ref[...])
    n_pad = h.shape[0]
    ri = lax.broadcasted_iota(jnp.int32, (n_pad, 1), 0)
    h = jnp.where(ri < n, h, 0.0)
    h_ref[...] = h
    pv = pv_ref[...]
    attn = jnp.sum(h * pv, axis=1, keepdims=True) / jnp.sqrt(
        jnp.sum(pv * pv))
    score = jnp.tanh(attn)
    score = jnp.where(ri < n, score, -2.0)
    s_ref[...] = score

    key = jax.lax.bitcast_convert_type(score, jnp.int32)
    key = jnp.where(key >= 0, key, key ^ jnp.int32(0x7FFFFFFF))
    cnt_nn = jnp.sum((key >= 0).astype(jnp.int32))
    lo = jnp.where(cnt_nn >= k, jnp.int32(0), jnp.int32(-2**31))
    hi = jnp.where(cnt_nn >= k, jnp.int32(2**31 - 1), jnp.int32(-1))

    def bs1(_, c):
        lo, hi = c
        mid = lo + (hi - lo) // 2
        pred = jnp.sum((key >= mid + 1).astype(jnp.int32)) >= k
        return (jnp.where(pred, mid + 1, lo), jnp.where(pred, hi, mid))

    lo, hi = lax.fori_loop(0, 31, bs1, (lo, hi))
    t = lo
    tie = key == t
    r = k - jnp.sum((key > t).astype(jnp.int32))

    def bs2(_, c):
        lo, hi = c
        mid = lo + (hi - lo) // 2
        pred = jnp.sum((tie & (ri <= mid)).astype(jnp.int32)) >= r
        return (jnp.where(pred, lo, mid + 1), jnp.where(pred, mid, hi))

    lo2, hi2 = lax.fori_loop(0, 14, bs2,
                             (jnp.int32(0), jnp.int32(n_pad - 1)))
    sel = (key > t) | (tie & (ri <= lo2))
    sel_ref[...] = sel.astype(jnp.int32)


def _post_pool(p0, p1, xw, dinv, b, pv, n, k):
    n_pad = p0.shape[0]
    return pl.pallas_call(
        functools.partial(_post_pool_body, n=n, k=k),
        out_shape=(
            jax.ShapeDtypeStruct((n_pad, _D), jnp.float32),
            jax.ShapeDtypeStruct((n_pad, 1), jnp.float32),
            jax.ShapeDtypeStruct((n_pad, 1), jnp.int32),
        ),
    )(p0, p1, xw, dinv, b.reshape(1, _D), pv.reshape(1, _D))


@functools.lru_cache(maxsize=None)
def _mesh():
    return plsc.VectorSubcoreMesh(core_axis_name="c", subcore_axis_name="s")


_GDN = lax.GatherDimensionNumbers(
    offset_dims=(), collapsed_slice_dims=(0,), start_index_map=(0,))


def _vtake(v, idx):
    return lax.gather(v, idx[:, None], _GDN, (1,),
                      mode=lax.GatherScatterMode.PROMISE_IN_BOUNDS)


def _vcumsum(v):
    iota = jax.lax.iota(jnp.int32, 16)
    for sh in (1, 2, 4, 8):
        idx = jnp.maximum(iota - sh, 0)
        v = v + jnp.where(iota >= sh, _vtake(v, idx), 0)
    return v


@functools.lru_cache(maxsize=None)
def _make_agg(n_pad):
    nchunks = _E_PER_TILE // _EBE
    rows_per_tile = n_pad // 16
    nz = rows_per_tile // 8

    @functools.partial(
        pl.kernel, mesh=_mesh(),
        out_type=(jax.ShapeDtypeStruct((n_pad, _D), jnp.float32),
                  jax.ShapeDtypeStruct((n_pad, _D), jnp.float32)),
        scratch_types=[
            pltpu.VMEM((_EBE,), jnp.int32),
            pltpu.VMEM((_EBE,), jnp.int32),
            pltpu.VMEM((_EBE, _D), jnp.float32),
            pltpu.VMEM((8, _D), jnp.float32),
            pltpu.VMEM_SHARED((n_pad, _D), jnp.float32),
            pltpu.SemaphoreType.DMA,
        ],
    )
    def agg(row_hbm, col_hbm, y_hbm, out0, out1, ridx, cidx, rows, zbuf,
            acc, sem):
        c = lax.axis_index("c")
        s = lax.axis_index("s")
        for i in range(8):
            for j in range(_D // 16):
                zbuf[i, pl.ds(j * 16, 16)] = jnp.zeros((16,), jnp.float32)
        r0 = s * rows_per_tile

        def zloop(i, carry):
            pltpu.sync_copy(zbuf, acc.at[pl.ds(r0 + i * 8, 8)])
            return carry

        lax.fori_loop(0, nz, zloop, 0)
        plsc.subcore_barrier()

        base = (c * 16 + s) * _E_PER_TILE

        def body(j, carry):
            off = base + j * _EBE
            pltpu.sync_copy(row_hbm.at[pl.ds(off, _EBE)], ridx)
            pltpu.sync_copy(col_hbm.at[pl.ds(off, _EBE)], cidx)
            pltpu.async_copy(y_hbm.at[ridx], rows, sem).wait()
            pltpu.sync_copy(rows, acc.at[cidx], add=True)
            return carry

        lax.fori_loop(0, nchunks, body, 0)
        plsc.subcore_barrier()

        sl = pl.ds(r0, rows_per_tile)

        @pl.when(c == 0)
        def _():
            pltpu.sync_copy(acc.at[sl], out0.at[sl])

        @pl.when(c == 1)
        def _():
            pltpu.sync_copy(acc.at[sl], out1.at[sl])

    return agg


@functools.lru_cache(maxsize=None)
def _make_deg0(n_pad):
    nchunks = _E_PER_TILE // _EBE
    chunk_m = n_pad // 16
    nzv = chunk_m // 16

    @functools.partial(
        pl.kernel, mesh=_mesh(),
        out_type=(jax.ShapeDtypeStruct((n_pad,), jnp.float32),
                  jax.ShapeDtypeStruct((n_pad,), jnp.float32)),
        scratch_types=[
            pltpu.VMEM((_EBE,), jnp.int32),
            pltpu.VMEM((_EBE,), jnp.float32),
            pltpu.VMEM((chunk_m,), jnp.float32),
            pltpu.VMEM_SHARED((n_pad,), jnp.float32),
        ],
    )
    def deg0(col_hbm, out0, out1, cidx, ones, zdeg, deg_sh):
        c = lax.axis_index("c")
        s = lax.axis_index("s")
        for v in range(_EBE // 16):
            ones[pl.ds(v * 16, 16)] = jnp.ones((16,), jnp.float32)
        for v in range(nzv):
            zdeg[pl.ds(v * 16, 16)] = jnp.zeros((16,), jnp.float32)
        mbase = s * chunk_m
        pltpu.sync_copy(zdeg, deg_sh.at[pl.ds(mbase, chunk_m)])
        plsc.subcore_barrier()

        base = (c * 16 + s) * _E_PER_TILE

        def body(j, carry):
            off = base + j * _EBE
            pltpu.sync_copy(col_hbm.at[pl.ds(off, _EBE)], cidx)
            pltpu.sync_copy(ones, deg_sh.at[cidx], add=True)
            return carry

        lax.fori_loop(0, nchunks, body, 0)
        plsc.subcore_barrier()

        sl = pl.ds(mbase, chunk_m)
        pltpu.sync_copy(deg_sh.at[sl], zdeg)

        @pl.when(c == 0)
        def _():
            pltpu.sync_copy(zdeg, out0.at[sl])

        @pl.when(c == 1)
        def _():
            pltpu.sync_copy(zdeg, out1.at[sl])

    return deg0


@functools.lru_cache(maxsize=None)
def _make_pool(np_prev, n_pad, k, n_prev):
    chunk_n = np_prev // 16
    nv = chunk_n // 16
    nv8 = chunk_n // _EB
    chunk_m = n_pad // 16
    nzv = chunk_m // 16
    rows_out = n_pad // 16
    npk = n_pad - k
    nchunks = _E_PER_TILE // _EBE

    @functools.partial(
        pl.kernel, mesh=_mesh(),
        compiler_params=pltpu.CompilerParams(needs_layout_passes=False),
        out_type=(
            jax.ShapeDtypeStruct((np_prev,), jnp.int32),
            jax.ShapeDtypeStruct((n_pad, _D), jnp.float32),
            jax.ShapeDtypeStruct((n_pad,), jnp.float32),
            jax.ShapeDtypeStruct((_E_PAD,), jnp.int32),
            jax.ShapeDtypeStruct((_E_PAD,), jnp.int32),
            jax.ShapeDtypeStruct((n_pad,), jnp.float32),
            jax.ShapeDtypeStruct((n_pad,), jnp.float32),
        ),
        scratch_types=[
            pltpu.VMEM((chunk_n,), jnp.int32),
            pltpu.VMEM((chunk_n,), jnp.float32),
            pltpu.VMEM((chunk_n,), jnp.int32),
            pltpu.VMEM((chunk_n,), jnp.int32),
            pltpu.VMEM((chunk_n,), jnp.float32),
            pltpu.VMEM((nv8, _EB), jnp.int32),
            pltpu.VMEM((np_prev,), jnp.int32),
            pltpu.VMEM((_EB, _D), jnp.float32),
            pltpu.VMEM((np_prev,), jnp.int32),
            pltpu.VMEM((_EBE,), jnp.int32),
            pltpu.VMEM((_EBE,), jnp.int32),
            pltpu.VMEM((_EBE,), jnp.int32),
            pltpu.VMEM((_EBE,), jnp.int32),
            pltpu.VMEM((_EBE,), jnp.float32),
            pltpu.VMEM((chunk_m,), jnp.float32),
            pltpu.VMEM((rows_out,), jnp.float32),
            pltpu.VMEM_SHARED((n_pad, _D), jnp.float32),
            pltpu.VMEM_SHARED((n_pad,), jnp.float32),
            pltpu.VMEM_SHARED((np_prev,), jnp.int32),
            pltpu.VMEM_SHARED((n_pad,), jnp.float32),
            pltpu.SemaphoreType.DMA,
        ],
    )
    def pool(sel_hbm, score_hbm, h_hbm, row_hbm, col_hbm,
             inv_out, xp_out, vals_out, nrow_out, ncol_out, dg0, dg1,
             selbuf, sbuf, invbuf, pbuf, svbuf, rankbuf, selall,
             hrows, inv_local, ebuf_r, ebuf_c, obuf_r, obuf_c, ones,
             zdeg, vbounce, xp_sh, vals_sh, inv_sh, deg_sh,
             sem):
        c = lax.axis_index("c")
        s = lax.axis_index("s")
        iota = jax.lax.iota(jnp.int32, 16)

        base = s * chunk_n
        pltpu.sync_copy(sel_hbm.at[pl.ds(base, chunk_n)], selbuf)
        pltpu.sync_copy(score_hbm.at[pl.ds(base, chunk_n)], sbuf)
        pltpu.sync_copy(sel_hbm, selall)
        fifteen = jnp.full((16,), 15, jnp.int32)

        off = jnp.zeros((16,), jnp.int32)
        run = jnp.zeros((16,), jnp.int32)
        for j in range(16):
            off = jnp.where(jnp.int32(j) == s, run, off)
            acc = jnp.zeros((16,), jnp.int32)
            for v in range(nv):
                acc = acc + selall[pl.ds(j * chunk_n + v * 16, 16)]
            run = run + _vtake(_vcumsum(acc), fifteen)

        for v in range(nzv):
            zdeg[pl.ds(v * 16, 16)] = jnp.zeros((16,), jnp.float32)
        mbase = s * chunk_m
        pltpu.sync_copy(zdeg, deg_sh.at[pl.ds(mbase, chunk_m)])
        for v in range(_EBE // 16):
            ones[pl.ds(v * 16, 16)] = jnp.ones((16,), jnp.float32)

        for v in range(nv):
            pbuf[pl.ds(v * 16, 16)] = jnp.full((16,), n_prev, jnp.int32)
        loc = jnp.zeros((16,), jnp.int32)
        for v in range(nv):
            selv = selbuf[pl.ds(v * 16, 16)]
            m = selv > 0
            cum = _vcumsum(selv)
            pos = loc + cum - 1
            invbuf[pl.ds(v * 16, 16)] = jnp.where(m, off + pos, k)
            gidx = base + v * 16 + iota
            plsc.store_scatter(pbuf, [pos], gidx, mask=m)
            plsc.store_scatter(svbuf, [pos], sbuf[pl.ds(v * 16, 16)],
                               mask=m)
            loc = loc + _vtake(cum, fifteen)
        for ci in range(nv8):
            for v in range(_EB // 16):
                jj = ci * _EB + v * 16 + iota
                rankbuf[ci, pl.ds(v * 16, 16)] = jnp.where(
                    jj < loc, off + jj, k)

        pltpu.sync_copy(invbuf, inv_sh.at[pl.ds(base, chunk_n)])

        @pl.when(c == 0)
        def _():
            pltpu.sync_copy(invbuf, inv_out.at[pl.ds(base, chunk_n)])

        for ci in range(nv8):
            pltpu.async_copy(h_hbm.at[pbuf.at[pl.ds(ci * _EB, _EB)]],
                             hrows, sem).wait()
            pltpu.sync_copy(hrows, xp_sh.at[rankbuf.at[ci]])
            pltpu.sync_copy(svbuf.at[pl.ds(ci * _EB, _EB)],
                            vals_sh.at[rankbuf.at[ci]])
        plsc.subcore_barrier()

        pltpu.sync_copy(inv_sh, inv_local)
        ebase = (c * 16 + s) * _E_PER_TILE

        def body(j, carry):
            off_e = ebase + j * _EBE
            pltpu.sync_copy(row_hbm.at[pl.ds(off_e, _EBE)], ebuf_r)
            pltpu.sync_copy(col_hbm.at[pl.ds(off_e, _EBE)], ebuf_c)
            for v in range(_EBE // 16):
                rv = ebuf_r[pl.ds(v * 16, 16)]
                cv = ebuf_c[pl.ds(v * 16, 16)]
                rr = plsc.load_gather(inv_local, [rv])
                cc = plsc.load_gather(inv_local, [cv])
                bad = (rr >= k) | (cc >= k)
                obuf_r[pl.ds(v * 16, 16)] = jnp.where(
                    bad, k + lax.rem(rv, jnp.int32(npk)), rr)
                obuf_c[pl.ds(v * 16, 16)] = jnp.where(
                    bad, k + lax.rem(cv, jnp.int32(npk)), cc)
            pltpu.sync_copy(obuf_r, nrow_out.at[pl.ds(off_e, _EBE)])
            pltpu.sync_copy(obuf_c, ncol_out.at[pl.ds(off_e, _EBE)])
            pltpu.sync_copy(ones, deg_sh.at[obuf_c], add=True)
            return carry

        lax.fori_loop(0, nchunks, body, 0)
        plsc.subcore_barrier()

        msl = pl.ds(mbase, chunk_m)
        pltpu.sync_copy(deg_sh.at[msl], zdeg)

        @pl.when(c == 0)
        def _():
            pltpu.sync_copy(zdeg, dg0.at[msl])
            rsl = pl.ds(s * rows_out, rows_out)
            pltpu.sync_copy(xp_sh.at[rsl], xp_out.at[rsl])
            pltpu.sync_copy(vals_sh.at[rsl], vbounce)
            pltpu.sync_copy(vbounce, vals_out.at[rsl])

        @pl.when(c == 1)
        def _():
            pltpu.sync_copy(zdeg, dg1.at[msl])

    return pool


@functools.lru_cache(maxsize=None)
def _make_unpool(out_pad, src_pad):
    rpt = out_pad // 32
    nchunks = rpt // _EB

    @functools.partial(
        pl.kernel, mesh=_mesh(),
        out_type=jax.ShapeDtypeStruct((out_pad, _D), jnp.float32),
        scratch_types=[
            pltpu.VMEM((_EB,), jnp.int32),
            pltpu.VMEM((_EB, _D), jnp.float32),
            pltpu.SemaphoreType.DMA,
        ],
    )
    def unpool(inv_hbm, src_hbm, out, ibuf, rows, sem):
        c = lax.axis_index("c")
        s = lax.axis_index("s")
        wbase = (c * 16 + s) * rpt

        def body(j, carry):
            off = wbase + j * _EB
            pltpu.sync_copy(inv_hbm.at[pl.ds(off, _EB)], ibuf)
            pltpu.async_copy(src_hbm.at[ibuf], rows, sem).wait()
            pltpu.sync_copy(rows, out.at[pl.ds(off, _EB)])
            return carry

        lax.fori_loop(0, nchunks, body, 0)

    return unpool


def kernel(x, edge_index, W_down0, b_down0, W_down1, b_down1, W_down2,
           b_down2, p_pool1, p_pool2, W_up1, b_up1, W_up2, b_up2):
    pad_idx = _N + jnp.arange(_E_PAD - _E, dtype=jnp.int32) % (_NP0 - _N)
    row = jnp.concatenate([edge_index[0], pad_idx])
    col = jnp.concatenate([edge_index[1], pad_idx])
    zz = (_STD / _SQRT_D) * jax.random.normal(
        jax.random.fold_in(jax.random.key(0), 777), (_N, _D), jnp.float32)

    d0a, d0b = _make_deg0(_NP0)(col)
    xw, y, dinv = _prep(x, None, W_down0, d0a, d0b, _N, _NP0, "x0")
    p0, p1 = _make_agg(_NP0)(row, col, y)
    h0, s1, sel1 = _post_pool(p0, p1, xw, dinv, b_down0, p_pool1, _N, _K1)

    inv1, xp1, vals1, r1, c1, d1a, d1b = _make_pool(_NP0, _NP1, _K1, _N)(
        sel1.reshape(_NP0), s1.reshape(_NP0), h0, row, col)
    xw, y, dinv1 = _prep(xp1, vals1, W_down1, d1a, d1b, _K1, _NP1, "pool")
    p0, p1 = _make_agg(_NP1)(r1, c1, y)
    h1, s2, sel2 = _post_pool(p0, p1, xw, dinv1, b_down1, p_pool2, _K1, _K2)

    inv2, xp2, vals2, r2, c2, d2a, d2b = _make_pool(_NP1, _NP2, _K2, _K1)(
        sel2.reshape(_NP1), s2.reshape(_NP1), h1, r1, c1)
    xw, y, dinv2 = _prep(xp2, vals2, W_down2, d2a, d2b, _K2, _NP2, "pool")
    p0, p1 = _make_agg(_NP2)(r2, c2, y)
    h2 = _post(p0, p1, xw, dinv2, b_down2, _K2, True)

    up1 = _make_unpool(_NP1, _NP2)(inv2, h2)
    xw, y, dinvu = _prep(h1, up1, W_up1, d1a, d1b, _K1, _NP1, "up")
    p0, p1 = _make_agg(_NP1)(r1, c1, y)
    hu1 = _post(p0, p1, xw, dinvu, b_up1, _K1, True)

    up0 = _make_unpool(_NP0, _NP1)(inv1, hu1)
    xw, y, dinvu = _prep(h0, up0, W_up2, d0a, d0b, _N, _NP0, "up")
    p0, p1 = _make_agg(_NP0)(row, col, y)
    return _post_final(p0, p1, xw, dinvu, b_up2, zz)


# speedup vs baseline: 15.5241x; 1.0860x over previous
import functools

import jax
import jax.numpy as jnp
from jax import lax
from jax.experimental import pallas as pl
from jax.experimental.pallas import tpu as pltpu
from jax.experimental.pallas import tpu_sc as plsc

_N = 10000
_E = 320000
_D = 128
_K1 = 5000
_K2 = 2500
_STD = 0.01
_SQRT_D = 0.1

_NP0 = 10240
_NP1 = 5120
_NP2 = 2560

_EB = 80
_E_PAD = 327680
_EBE = 128
_E_PER_TILE = _E_PAD // 32


def _prep_body(a_ref, b_ref, w_ref, d0_ref, d1_ref, xw_ref, y_ref,
               dinv_ref, *, n, mode):
    if mode == "x0":
        xin = a_ref[...]
    elif mode == "up":
        xin = a_ref[...] + b_ref[...]
    else:
        xin = a_ref[...]
    xw = jnp.dot(xin, w_ref[...], preferred_element_type=jnp.float32)
    if mode == "pool":
        xw = xw * b_ref[...]
    dinv = jax.lax.rsqrt(d0_ref[...] + d1_ref[...] + 2.0)
    n_pad = dinv_ref.shape[0]
    if mode == "x0":
        xw_ref[:n, :] = xw
        xw_ref[n:, :] = jnp.zeros((n_pad - n, _D), jnp.float32)
        y_ref[:n, :] = xw * dinv[:n]
        y_ref[n:, :] = jnp.zeros((n_pad - n, _D), jnp.float32)
    else:
        ri = lax.broadcasted_iota(jnp.int32, (n_pad, 1), 0)
        xw = jnp.where(ri < n, xw, 0.0)
        xw_ref[...] = xw
        y_ref[...] = xw * dinv
    dinv_ref[...] = dinv


def _prep(a, b, w, d0, d1, n, n_pad, mode):
    body = functools.partial(_prep_body, n=n, mode=mode)
    args = [a]
    if mode == "up":
        args.append(b)
    elif mode == "pool":
        args.append(b.reshape(n_pad, 1))
    else:
        args.append(jnp.zeros((1, 1), jnp.float32))
    args += [w, d0.reshape(n_pad, 1), d1.reshape(n_pad, 1)]
    return pl.pallas_call(
        body,
        out_shape=(
            jax.ShapeDtypeStruct((n_pad, _D), jnp.float32),
            jax.ShapeDtypeStruct((n_pad, _D), jnp.float32),
            jax.ShapeDtypeStruct((n_pad, 1), jnp.float32),
        ),
    )(*args)


def _post_body(p0_ref, p1_ref, xw_ref, dinv_ref, b_ref, o_ref, *, n,
               do_tanh):
    dinv = dinv_ref[...]
    o = ((p0_ref[...] + p1_ref[...]) * dinv
         + 2.0 * dinv * dinv * xw_ref[...] + b_ref[...])
    if do_tanh:
        o = jnp.tanh(o)
    ri = lax.broadcasted_iota(jnp.int32, o.shape, 0)
    o_ref[...] = jnp.where(ri < n, o, 0.0)


def _post(p0, p1, xw, dinv, b, n, do_tanh):
    n_pad = p0.shape[0]
    return pl.pallas_call(
        functools.partial(_post_body, n=n, do_tanh=do_tanh),
        out_shape=jax.ShapeDtypeStruct((n_pad, _D), jnp.float32),
    )(p0, p1, xw, dinv, b.reshape(1, _D))


def _post_final_body(p0_ref, p1_ref, xw_ref, dinv_ref, b_ref, z_ref, o_ref):
    dinv = dinv_ref[...]
    o = ((p0_ref[...] + p1_ref[...]) * dinv
         + 2.0 * dinv * dinv * xw_ref[...] + b_ref[...])
    o_ref[...] = o[:_N, :] + z_ref[...]


def _post_final(p0, p1, xw, dinv, b, z):
    return pl.pallas_call(
        _post_final_body,
        out_shape=jax.ShapeDtypeStruct((_N, _D), jnp.float32),
    )(p0, p1, xw, dinv, b.reshape(1, _D), z)


def _post_pool_body(p0_ref, p1_ref, xw_ref, dinv_ref, b_ref, pv_ref,
                    h_ref, s_ref, sel_ref, *, n, k):
    dinv = dinv_ref[...]
    h = jnp.tanh((p0_ref[...] + p1_ref[...]) * dinv
                 + 2.0 * dinv * dinv * xw_ref[...] + b_ref[...])
    n_pad = h.shape[0]
    ri = lax.broadcasted_iota(jnp.int32, (n_pad, 1), 0)
    h = jnp.where(ri < n, h, 0.0)
    h_ref[...] = h
    pv = pv_ref[...]
    attn = jnp.sum(h * pv, axis=1, keepdims=True) / jnp.sqrt(
        jnp.sum(pv * pv))
    score = jnp.tanh(attn)
    score = jnp.where(ri < n, score, -2.0)
    s_ref[...] = score

    key = jax.lax.bitcast_convert_type(score, jnp.int32)
    key = jnp.where(key >= 0, key, key ^ jnp.int32(0x7FFFFFFF))
    cnt_nn = jnp.sum((key >= 0).astype(jnp.int32))
    lo = jnp.where(cnt_nn >= k, jnp.int32(0), jnp.int32(-2**31))
    hi = jnp.where(cnt_nn >= k, jnp.int32(2**31 - 1), jnp.int32(-1))

    def bs1(_, c):
        lo, hi = c
        mid = lo + (hi - lo) // 2
        pred = jnp.sum((key >= mid + 1).astype(jnp.int32)) >= k
        return (jnp.where(pred, mid + 1, lo), jnp.where(pred, hi, mid))

    lo, hi = lax.fori_loop(0, 31, bs1, (lo, hi))
    t = lo
    tie = key == t
    r = k - jnp.sum((key > t).astype(jnp.int32))

    def bs2(_, c):
        lo, hi = c
        mid = lo + (hi - lo) // 2
        pred = jnp.sum((tie & (ri <= mid)).astype(jnp.int32)) >= r
        return (jnp.where(pred, lo, mid + 1), jnp.where(pred, mid, hi))

    lo2, hi2 = lax.fori_loop(0, 14, bs2,
                             (jnp.int32(0), jnp.int32(n_pad - 1)))
    sel = (key > t) | (tie & (ri <= lo2))
    sel_ref[...] = sel.astype(jnp.int32)


def _post_pool(p0, p1, xw, dinv, b, pv, n, k):
    n_pad = p0.shape[0]
    return pl.pallas_call(
        functools.partial(_post_pool_body, n=n, k=k),
        out_shape=(
            jax.ShapeDtypeStruct((n_pad, _D), jnp.float32),
            jax.ShapeDtypeStruct((n_pad, 1), jnp.float32),
            jax.ShapeDtypeStruct((n_pad, 1), jnp.int32),
        ),
    )(p0, p1, xw, dinv, b.reshape(1, _D), pv.reshape(1, _D))


@functools.lru_cache(maxsize=None)
def _mesh():
    return plsc.VectorSubcoreMesh(core_axis_name="c", subcore_axis_name="s")


_GDN = lax.GatherDimensionNumbers(
    offset_dims=(), collapsed_slice_dims=(0,), start_index_map=(0,))


def _vtake(v, idx):
    return lax.gather(v, idx[:, None], _GDN, (1,),
                      mode=lax.GatherScatterMode.PROMISE_IN_BOUNDS)


def _vcumsum(v):
    iota = jax.lax.iota(jnp.int32, 16)
    for sh in (1, 2, 4, 8):
        idx = jnp.maximum(iota - sh, 0)
        v = v + jnp.where(iota >= sh, _vtake(v, idx), 0)
    return v


@functools.lru_cache(maxsize=None)
def _make_agg(n_pad):
    nchunks = _E_PER_TILE // _EBE
    nsuper = nchunks // 4
    rows_per_tile = n_pad // 16
    nz = rows_per_tile // 8

    @functools.partial(
        pl.kernel, mesh=_mesh(),
        out_type=(jax.ShapeDtypeStruct((n_pad, _D), jnp.float32),
                  jax.ShapeDtypeStruct((n_pad, _D), jnp.float32)),
        scratch_types=[
            pltpu.VMEM((4, _EBE), jnp.int32),
            pltpu.VMEM((4, _EBE), jnp.int32),
            pltpu.VMEM((_EBE, _D), jnp.float32),
            pltpu.VMEM((8, _D), jnp.float32),
            pltpu.VMEM_SHARED((n_pad, _D), jnp.float32),
            pltpu.SemaphoreType.DMA,
        ],
    )
    def agg(row_hbm, col_hbm, y_hbm, out0, out1, ridx, cidx, rows, zbuf,
            acc, sem):
        c = lax.axis_index("c")
        s = lax.axis_index("s")
        for i in range(8):
            for j in range(_D // 16):
                zbuf[i, pl.ds(j * 16, 16)] = jnp.zeros((16,), jnp.float32)
        r0 = s * rows_per_tile

        def zloop(i, carry):
            pltpu.sync_copy(zbuf, acc.at[pl.ds(r0 + i * 8, 8)])
            return carry

        lax.fori_loop(0, nz, zloop, 0)
        plsc.subcore_barrier()

        base = (c * 16 + s) * nchunks

        def body(j, carry):
            off = base + j * 4
            pltpu.sync_copy(row_hbm.at[pl.ds(off, 4)], ridx)
            pltpu.sync_copy(col_hbm.at[pl.ds(off, 4)], cidx)
            for i in range(4):
                pltpu.async_copy(y_hbm.at[ridx.at[i]], rows, sem).wait()
                pltpu.sync_copy(rows, acc.at[cidx.at[i]], add=True)
            return carry

        lax.fori_loop(0, nsuper, body, 0)
        plsc.subcore_barrier()

        sl = pl.ds(r0, rows_per_tile)

        @pl.when(c == 0)
        def _():
            pltpu.sync_copy(acc.at[sl], out0.at[sl])

        @pl.when(c == 1)
        def _():
            pltpu.sync_copy(acc.at[sl], out1.at[sl])

    return agg


@functools.lru_cache(maxsize=None)
def _make_deg0(n_pad):
    nchunks = _E_PER_TILE // _EBE
    chunk_m = n_pad // 16
    nzv = chunk_m // 16

    @functools.partial(
        pl.kernel, mesh=_mesh(),
        out_type=(jax.ShapeDtypeStruct((n_pad,), jnp.float32),
                  jax.ShapeDtypeStruct((n_pad,), jnp.float32)),
        scratch_types=[
            pltpu.VMEM((_EBE,), jnp.int32),
            pltpu.VMEM((_EBE,), jnp.float32),
            pltpu.VMEM((chunk_m,), jnp.float32),
            pltpu.VMEM_SHARED((n_pad,), jnp.float32),
        ],
    )
    def deg0(col_hbm, out0, out1, cidx, ones, zdeg, deg_sh):
        c = lax.axis_index("c")
        s = lax.axis_index("s")
        for v in range(_EBE // 16):
            ones[pl.ds(v * 16, 16)] = jnp.ones((16,), jnp.float32)
        for v in range(nzv):
            zdeg[pl.ds(v * 16, 16)] = jnp.zeros((16,), jnp.float32)
        mbase = s * chunk_m
        pltpu.sync_copy(zdeg, deg_sh.at[pl.ds(mbase, chunk_m)])
        plsc.subcore_barrier()

        base = (c * 16 + s) * _E_PER_TILE

        def body(j, carry):
            off = base + j * _EBE
            pltpu.sync_copy(col_hbm.at[pl.ds(off, _EBE)], cidx)
            pltpu.sync_copy(ones, deg_sh.at[cidx], add=True)
            return carry

        lax.fori_loop(0, nchunks, body, 0)
        plsc.subcore_barrier()

        sl = pl.ds(mbase, chunk_m)
        pltpu.sync_copy(deg_sh.at[sl], zdeg)

        @pl.when(c == 0)
        def _():
            pltpu.sync_copy(zdeg, out0.at[sl])

        @pl.when(c == 1)
        def _():
            pltpu.sync_copy(zdeg, out1.at[sl])

    return deg0


@functools.lru_cache(maxsize=None)
def _make_pool(np_prev, n_pad, k, n_prev):
    chunk_n = np_prev // 16
    nv = chunk_n // 16
    nv8 = chunk_n // _EB
    chunk_m = n_pad // 16
    nzv = chunk_m // 16
    rows_out = n_pad // 16
    npk = n_pad - k
    nchunks = _E_PER_TILE // _EBE

    @functools.partial(
        pl.kernel, mesh=_mesh(),
        compiler_params=pltpu.CompilerParams(needs_layout_passes=False),
        out_type=(
            jax.ShapeDtypeStruct((np_prev,), jnp.int32),
            jax.ShapeDtypeStruct((n_pad, _D), jnp.float32),
            jax.ShapeDtypeStruct((n_pad,), jnp.float32),
            jax.ShapeDtypeStruct((_E_PAD,), jnp.int32),
            jax.ShapeDtypeStruct((_E_PAD,), jnp.int32),
            jax.ShapeDtypeStruct((n_pad,), jnp.float32),
            jax.ShapeDtypeStruct((n_pad,), jnp.float32),
        ),
        scratch_types=[
            pltpu.VMEM((chunk_n,), jnp.int32),
            pltpu.VMEM((chunk_n,), jnp.float32),
            pltpu.VMEM((chunk_n,), jnp.int32),
            pltpu.VMEM((chunk_n,), jnp.int32),
            pltpu.VMEM((chunk_n,), jnp.float32),
            pltpu.VMEM((nv8, _EB), jnp.int32),
            pltpu.VMEM((np_prev,), jnp.int32),
            pltpu.VMEM((_EB, _D), jnp.float32),
            pltpu.VMEM((np_prev,), jnp.int32),
            pltpu.VMEM((_EBE,), jnp.int32),
            pltpu.VMEM((_EBE,), jnp.int32),
            pltpu.VMEM((_EBE,), jnp.int32),
            pltpu.VMEM((_EBE,), jnp.int32),
            pltpu.VMEM((_EBE,), jnp.float32),
            pltpu.VMEM((chunk_m,), jnp.float32),
            pltpu.VMEM((rows_out,), jnp.float32),
            pltpu.VMEM_SHARED((n_pad, _D), jnp.float32),
            pltpu.VMEM_SHARED((n_pad,), jnp.float32),
            pltpu.VMEM_SHARED((np_prev,), jnp.int32),
            pltpu.VMEM_SHARED((n_pad,), jnp.float32),
            pltpu.SemaphoreType.DMA,
        ],
    )
    def pool(sel_hbm, score_hbm, h_hbm, row_hbm, col_hbm,
             inv_out, xp_out, vals_out, nrow_out, ncol_out, dg0, dg1,
             selbuf, sbuf, invbuf, pbuf, svbuf, rankbuf, selall,
             hrows, inv_local, ebuf_r, ebuf_c, obuf_r, obuf_c, ones,
             zdeg, vbounce, xp_sh, vals_sh, inv_sh, deg_sh,
             sem):
        c = lax.axis_index("c")
        s = lax.axis_index("s")
        iota = jax.lax.iota(jnp.int32, 16)

        base = s * chunk_n
        pltpu.sync_copy(sel_hbm.at[pl.ds(base, chunk_n)], selbuf)
        pltpu.sync_copy(score_hbm.at[pl.ds(base, chunk_n)], sbuf)
        pltpu.sync_copy(sel_hbm, selall)
        fifteen = jnp.full((16,), 15, jnp.int32)

        off = jnp.zeros((16,), jnp.int32)
        run = jnp.zeros((16,), jnp.int32)
        for j in range(16):
            off = jnp.where(jnp.int32(j) == s, run, off)
            acc = jnp.zeros((16,), jnp.int32)
            for v in range(nv):
                acc = acc + selall[pl.ds(j * chunk_n + v * 16, 16)]
            run = run + _vtake(_vcumsum(acc), fifteen)

        for v in range(nzv):
            zdeg[pl.ds(v * 16, 16)] = jnp.zeros((16,), jnp.float32)
        mbase = s * chunk_m
        pltpu.sync_copy(zdeg, deg_sh.at[pl.ds(mbase, chunk_m)])
        for v in range(_EBE // 16):
            ones[pl.ds(v * 16, 16)] = jnp.ones((16,), jnp.float32)

        for v in range(nv):
            pbuf[pl.ds(v * 16, 16)] = jnp.full((16,), n_prev, jnp.int32)
        loc = jnp.zeros((16,), jnp.int32)
        for v in range(nv):
            selv = selbuf[pl.ds(v * 16, 16)]
            m = selv > 0
            cum = _vcumsum(selv)
            pos = loc + cum - 1
            invbuf[pl.ds(v * 16, 16)] = jnp.where(m, off + pos, k)
            gidx = base + v * 16 + iota
            plsc.store_scatter(pbuf, [pos], gidx, mask=m)
            plsc.store_scatter(svbuf, [pos], sbuf[pl.ds(v * 16, 16)],
                               mask=m)
            loc = loc + _vtake(cum, fifteen)
        for ci in range(nv8):
            for v in range(_EB // 16):
                jj = ci * _EB + v * 16 + iota
                rankbuf[ci, pl.ds(v * 16, 16)] = jnp.where(
                    jj < loc, off + jj, k)

        pltpu.sync_copy(invbuf, inv_sh.at[pl.ds(base, chunk_n)])

        @pl.when(c == 0)
        def _():
            pltpu.sync_copy(invbuf, inv_out.at[pl.ds(base, chunk_n)])

        for ci in range(nv8):
            pltpu.async_copy(h_hbm.at[pbuf.at[pl.ds(ci * _EB, _EB)]],
                             hrows, sem).wait()
            pltpu.sync_copy(hrows, xp_sh.at[rankbuf.at[ci]])
            pltpu.sync_copy(svbuf.at[pl.ds(ci * _EB, _EB)],
                            vals_sh.at[rankbuf.at[ci]])
        plsc.subcore_barrier()

        pltpu.sync_copy(inv_sh, inv_local)
        ebase = (c * 16 + s) * _E_PER_TILE

        def body(j, carry):
            off_e = ebase + j * _EBE
            pltpu.sync_copy(row_hbm.at[pl.ds(off_e, _EBE)], ebuf_r)
            pltpu.sync_copy(col_hbm.at[pl.ds(off_e, _EBE)], ebuf_c)
            for v in range(_EBE // 16):
                rv = ebuf_r[pl.ds(v * 16, 16)]
                cv = ebuf_c[pl.ds(v * 16, 16)]
                rr = plsc.load_gather(inv_local, [rv])
                cc = plsc.load_gather(inv_local, [cv])
                bad = (rr >= k) | (cc >= k)
                obuf_r[pl.ds(v * 16, 16)] = jnp.where(
                    bad, k + lax.rem(rv, jnp.int32(npk)), rr)
                obuf_c[pl.ds(v * 16, 16)] = jnp.where(
                    bad, k + lax.rem(cv, jnp.int32(npk)), cc)
            pltpu.sync_copy(obuf_r, nrow_out.at[pl.ds(off_e, _EBE)])
            pltpu.sync_copy(obuf_c, ncol_out.at[pl.ds(off_e, _EBE)])
            pltpu.sync_copy(ones, deg_sh.at[obuf_c], add=True)
            return carry

        lax.fori_loop(0, nchunks, body, 0)
        plsc.subcore_barrier()

        msl = pl.ds(mbase, chunk_m)
        pltpu.sync_copy(deg_sh.at[msl], zdeg)

        @pl.when(c == 0)
        def _():
            pltpu.sync_copy(zdeg, dg0.at[msl])
            rsl = pl.ds(s * rows_out, rows_out)
            pltpu.sync_copy(xp_sh.at[rsl], xp_out.at[rsl])
            pltpu.sync_copy(vals_sh.at[rsl], vbounce)
            pltpu.sync_copy(vbounce, vals_out.at[rsl])

        @pl.when(c == 1)
        def _():
            pltpu.sync_copy(zdeg, dg1.at[msl])

    return pool


@functools.lru_cache(maxsize=None)
def _make_unpool(out_pad, src_pad):
    rpt = out_pad // 32
    nchunks = rpt // _EB

    @functools.partial(
        pl.kernel, mesh=_mesh(),
        out_type=jax.ShapeDtypeStruct((out_pad, _D), jnp.float32),
        scratch_types=[
            pltpu.VMEM((_EB,), jnp.int32),
            pltpu.VMEM((_EB, _D), jnp.float32),
            pltpu.SemaphoreType.DMA,
        ],
    )
    def unpool(inv_hbm, src_hbm, out, ibuf, rows, sem):
        c = lax.axis_index("c")
        s = lax.axis_index("s")
        wbase = (c * 16 + s) * rpt

        def body(j, carry):
            off = wbase + j * _EB
            pltpu.sync_copy(inv_hbm.at[pl.ds(off, _EB)], ibuf)
            pltpu.async_copy(src_hbm.at[ibuf], rows, sem).wait()
            pltpu.sync_copy(rows, out.at[pl.ds(off, _EB)])
            return carry

        lax.fori_loop(0, nchunks, body, 0)

    return unpool


def kernel(x, edge_index, W_down0, b_down0, W_down1, b_down1, W_down2,
           b_down2, p_pool1, p_pool2, W_up1, b_up1, W_up2, b_up2):
    pad_idx = _N + jnp.arange(_E_PAD - _E, dtype=jnp.int32) % (_NP0 - _N)
    row = jnp.concatenate([edge_index[0], pad_idx])
    col = jnp.concatenate([edge_index[1], pad_idx])
    zz = (_STD / _SQRT_D) * jax.random.normal(
        jax.random.fold_in(jax.random.key(0), 777), (_N, _D), jnp.float32)

    d0a, d0b = _make_deg0(_NP0)(col)
    xw, y, dinv = _prep(x, None, W_down0, d0a, d0b, _N, _NP0, "x0")
    p0, p1 = _make_agg(_NP0)(row.reshape(-1, _EBE), col.reshape(-1, _EBE), y)
    h0, s1, sel1 = _post_pool(p0, p1, xw, dinv, b_down0, p_pool1, _N, _K1)

    inv1, xp1, vals1, r1, c1, d1a, d1b = _make_pool(_NP0, _NP1, _K1, _N)(
        sel1.reshape(_NP0), s1.reshape(_NP0), h0, row, col)
    xw, y, dinv1 = _prep(xp1, vals1, W_down1, d1a, d1b, _K1, _NP1, "pool")
    p0, p1 = _make_agg(_NP1)(r1.reshape(-1, _EBE), c1.reshape(-1, _EBE), y)
    h1, s2, sel2 = _post_pool(p0, p1, xw, dinv1, b_down1, p_pool2, _K1, _K2)

    inv2, xp2, vals2, r2, c2, d2a, d2b = _make_pool(_NP1, _NP2, _K2, _K1)(
        sel2.reshape(_NP1), s2.reshape(_NP1), h1, r1, c1)
    xw, y, dinv2 = _prep(xp2, vals2, W_down2, d2a, d2b, _K2, _NP2, "pool")
    p0, p1 = _make_agg(_NP2)(r2.reshape(-1, _EBE), c2.reshape(-1, _EBE), y)
    h2 = _post(p0, p1, xw, dinv2, b_down2, _K2, True)

    up1 = _make_unpool(_NP1, _NP2)(inv2, h2)
    xw, y, dinvu = _prep(h1, up1, W_up1, d1a, d1b, _K1, _NP1, "up")
    p0, p1 = _make_agg(_NP1)(r1.reshape(-1, _EBE), c1.reshape(-1, _EBE), y)
    hu1 = _post(p0, p1, xw, dinvu, b_up1, _K1, True)

    up0 = _make_unpool(_NP0, _NP1)(inv1, hu1)
    xw, y, dinvu = _prep(h0, up0, W_up2, d0a, d0b, _N, _NP0, "up")
    p0, p1 = _make_agg(_NP0)(row.reshape(-1, _EBE), col.reshape(-1, _EBE), y)
    return _post_final(p0, p1, xw, dinvu, b_up2, zz)
